# Initial kernel scaffold; baseline (speedup 1.0000x reference)
import functools

import jax
import jax.numpy as jnp
from jax import lax
from jax.experimental import pallas as pl
from jax.experimental.pallas import tpu as pltpu
from jax.experimental.pallas import tpu_sc as plsc

NC = 2
NS = 16
LN = 16

N = 20000
D = 128
DH = 64
NNZ = 320000
EB = 512
NNZ_PAD = 327680
G_PAD = 20480

_f32 = jnp.float32
_i32 = jnp.int32


def _make_spmm(relu, has_init):
    MS = N // NS
    NCH = (NNZ_PAD // NS) // EB

    mesh = plsc.VectorSubcoreMesh(core_axis_name="c", subcore_axis_name="s")
    scratch = [
        pltpu.VMEM((EB // 128, 128), _i32),
        pltpu.VMEM((EB // 128, 128), _i32),
        pltpu.VMEM((EB,), _f32),
        pltpu.VMEM((EB, DH), _f32),
        pltpu.VMEM((250, DH), _f32),
        pltpu.VMEM_SHARED((N, DH), _f32),
        pltpu.SemaphoreType.DMA,
    ]

    def body(*refs):
        if has_init:
            (x2, rows2, cols2, vals, init, out,
             idxbuf, rowbuf, valbuf, gbuf, obuf, acc, sem) = refs
        else:
            (x2, rows2, cols2, vals, out,
             idxbuf, rowbuf, valbuf, gbuf, obuf, acc, sem) = refs
        c = lax.axis_index("c")
        s = lax.axis_index("s")
        r0 = s * MS

        if has_init:
            pltpu.sync_copy(init.at[pl.ds(r0, MS), c], acc.at[pl.ds(r0, MS)])
        else:
            def zrow(j, _):
                for q in range(4):
                    obuf[j, pl.ds(q * LN, LN)] = jnp.zeros((LN,), _f32)
                return 0
            lax.fori_loop(0, 250, zrow, 0, unroll=8)
            for oi in range(MS // 250):
                pltpu.sync_copy(obuf, acc.at[pl.ds(r0 + oi * 250, 250)])
        plsc.subcore_barrier()

        e128_0 = s * ((NNZ_PAD // NS) // 128)

        def chunk(i, _):
            b128 = e128_0 + i * (EB // 128)
            eb = (s * (NNZ_PAD // NS)) + i * EB
            pltpu.sync_copy(cols2.at[pl.ds(b128, EB // 128)], idxbuf)
            pltpu.sync_copy(rows2.at[pl.ds(b128, EB // 128)], rowbuf)
            pltpu.sync_copy(vals.at[pl.ds(eb, EB)], valbuf)
            def xform(j, _):
                for q in range(8):
                    sl = pl.ds(q * LN, LN)
                    idxbuf[j, sl] = idxbuf[j, sl] * 2 + c
                return 0
            lax.fori_loop(0, EB // 128, xform, 0, unroll=True)
            descs = [
                pltpu.async_copy(x2.at[idxbuf.at[q]],
                                 gbuf.at[pl.ds(q * 128, 128)], sem)
                for q in range(EB // 128)
            ]
            for dd in descs:
                dd.wait()
            def scale(g, _):
                b16 = g * LN
                v16 = valbuf[pl.ds(b16, LN)]
                for jj in range(LN):
                    sp = jnp.broadcast_to(v16[jj], (LN,))
                    e = b16 + jj
                    for q in range(4):
                        sl = pl.ds(q * LN, LN)
                        gbuf[e, sl] = gbuf[e, sl] * sp
                return 0
            lax.fori_loop(0, EB // LN, scale, 0)
            for q in range(EB // 128):
                pltpu.sync_copy(gbuf.at[pl.ds(q * 128, 128)],
                                acc.at[rowbuf.at[q]], add=True)
            return 0

        lax.fori_loop(0, NCH, chunk, 0)
        plsc.subcore_barrier()

        for oi in range(MS // 250):
            ro = r0 + oi * 250
            pltpu.sync_copy(acc.at[pl.ds(ro, 250)], obuf)
            if relu:
                def rl(j, _):
                    for q in range(4):
                        sl = pl.ds(q * LN, LN)
                        obuf[j, sl] = jnp.maximum(obuf[j, sl], 0.0)
                    return 0
                lax.fori_loop(0, 250, rl, 0, unroll=8)
            pltpu.sync_copy(obuf, out.at[pl.ds(ro, 250), c])

    return pl.kernel(
        body,
        out_type=jax.ShapeDtypeStruct((N, 2, DH), _f32),
        mesh=mesh,
        scratch_types=scratch,
    )


def _make_gather():
    mesh = plsc.VectorSubcoreMesh(core_axis_name="c", subcore_axis_name="s")
    scratch = [
        pltpu.VMEM((50000,), _i32),
        pltpu.VMEM((128,), _i32),
        pltpu.VMEM((128,), _i32),
        pltpu.VMEM((128, D), _f32),
        pltpu.SemaphoreType.DMA,
    ]

    def body(E, dcol, gv2, sf, df, dvm, idxbuf, idx2buf, gbuf, sem):
        c = lax.axis_index("c")
        s = lax.axis_index("s")
        w = c * NS + s
        pltpu.sync_copy(dcol, dvm)

        def chunk(i, _):
            row128 = w * 5 + i
            pltpu.sync_copy(gv2.at[row128], idxbuf)
            pltpu.async_copy(E.at[idxbuf], gbuf, sem).wait()
            pltpu.sync_copy(gbuf, sf.at[pl.ds(row128 * 128, 128)])
            for j in range(8):
                sl = pl.ds(j * LN, LN)
                idx2buf[sl] = plsc.load_gather(dvm, [idxbuf[sl]])
            pltpu.async_copy(E.at[idx2buf], gbuf, sem).wait()
            pltpu.sync_copy(gbuf, df.at[pl.ds(row128 * 128, 128)])
            return 0

        lax.fori_loop(0, 5, chunk, 0)

    return pl.kernel(
        body,
        out_type=(jax.ShapeDtypeStruct((G_PAD, D), _f32),
                  jax.ShapeDtypeStruct((G_PAD, D), _f32)),
        mesh=mesh,
        scratch_types=scratch,
    )


_R = 500


def _fusion_body(sf, df, w1, b1, w2, hg, out):
    a = sf[...]
    b = df[...]
    w1v = w1[...]
    b1v = b1[...]
    w2v = w2[...]
    h0 = jnp.tanh(jnp.dot(a, w1v, preferred_element_type=_f32) + b1v)
    h1 = jnp.tanh(jnp.dot(b, w1v, preferred_element_type=_f32) + b1v)
    s0 = jnp.sum(h0 * w2v.T, axis=1, keepdims=True)
    s1 = jnp.sum(h1 * w2v.T, axis=1, keepdims=True)
    p = jax.nn.sigmoid(s0 - s1)
    fused = p * a + (1.0 - p) * b
    out[...] = jnp.dot(fused, hg[...], preferred_element_type=_f32)


def _make_fusion():
    grid = (N // _R,)
    row = pl.BlockSpec((_R, D), lambda i: (i, 0))
    full = pl.BlockSpec((D, D), lambda i: (0, 0))
    return pl.pallas_call(
        _fusion_body,
        grid=grid,
        in_specs=[row, row, full,
                  pl.BlockSpec((1, D), lambda i: (0, 0)),
                  pl.BlockSpec((D, 1), lambda i: (0, 0)),
                  full],
        out_specs=row,
        out_shape=jax.ShapeDtypeStruct((N, D), _f32),
    )


def _make_mm(relu_x, bias, ysum):
    grid = (N // _R,)
    row = pl.BlockSpec((_R, D), lambda i: (i, 0))
    full = pl.BlockSpec((D, D), lambda i: (0, 0))
    brow = pl.BlockSpec((1, D), lambda i: (0, 0))

    def mm_body(*refs):
        i = 0
        x = refs[i][...]; i += 1
        w = refs[i][...]; i += 1
        bv = None
        if bias:
            bv = refs[i][...]; i += 1
        yv = None
        if ysum:
            yv = jnp.maximum(refs[i][...], 0.0); i += 1
        if relu_x:
            x = jnp.maximum(x, 0.0)
        o = jnp.dot(x, w, preferred_element_type=_f32)
        if bias:
            o = o + bv
        refs[i][...] = o
        if ysum:
            refs[i + 1][...] = yv + x

    in_specs = [row, full]
    if bias:
        in_specs.append(brow)
    if ysum:
        in_specs.append(row)
    out_shape = jax.ShapeDtypeStruct((N, D), _f32)
    if ysum:
        return pl.pallas_call(mm_body, grid=grid, in_specs=in_specs,
                              out_specs=(row, row),
                              out_shape=(out_shape, out_shape))
    return pl.pallas_call(mm_body, grid=grid, in_specs=in_specs,
                          out_specs=row, out_shape=out_shape)


def _pad_edges(rows, cols, vals):
    pad = NNZ_PAD - NNZ
    rows2 = jnp.concatenate([rows, jnp.zeros((pad,), _i32)]).reshape(-1, 128)
    cols2 = jnp.concatenate([cols, jnp.zeros((pad,), _i32)]).reshape(-1, 128)
    valsp = jnp.concatenate([vals, jnp.zeros((pad,), _f32)])
    return rows2, cols2, valsp


def kernel(static_embeddings, fusion_W1, fusion_b1, fusion_W2, fusion_b2,
           hgcn_W, lg_W, lg_b,
           coef1_vals, coef2_vals, lg_vals,
           guid_v, duid_trace_v,
           coef1_rows, coef1_cols, coef2_rows, coef2_cols, lg_rows, lg_cols):
    E = static_embeddings
    T = guid_v.shape[0]

    spmm = _make_spmm(relu=False, has_init=False)
    spmm_relu = _make_spmm(relu=True, has_init=False)
    spmm_init = _make_spmm(relu=False, has_init=True)
    gather = _make_gather()
    fusion = _make_fusion()
    mm_relu = _make_mm(relu_x=True, bias=False, ysum=False)
    mm_bias_relu = _make_mm(relu_x=True, bias=True, ysum=False)
    mm_bias_ysum = _make_mm(relu_x=False, bias=True, ysum=True)

    def as2(x):
        return x.reshape(2 * N, DH)

    def as3(x):
        return x.reshape(N, 2, DH)

    def flat(x):
        return x.reshape(N, D)

    dyn_v_parts = [E]
    dyn_u_parts = [E]
    for t in range(T):
        gvp = jnp.concatenate(
            [guid_v[t], jnp.zeros((G_PAD - N,), _i32)]).reshape(-1, 128)
        dcol = duid_trace_v[:, t]
        sfp, dfp = gather(E, dcol, gvp)
        sf, df = sfp[:N], dfp[:N]

        x0 = fusion(sf, df, fusion_W1[t], fusion_b1[t].reshape(1, D),
                    fusion_W2[t], hgcn_W[t, 0])

        c1r, c1c, c1v = _pad_edges(coef1_rows[t], coef1_cols[t], coef1_vals[t])
        c2r, c2c, c2v = _pad_edges(coef2_rows[t], coef2_cols[t], coef2_vals[t])
        lgr, lgc, lgv = _pad_edges(lg_rows[t], lg_cols[t], lg_vals[t])

        x1 = spmm(as2(x0), c1r, c1c, c1v)
        x1m = mm_relu(flat(x1), hgcn_W[t, 1])
        x2 = spmm(as2(x1m), c1r, c1c, c1v)
        x2m = mm_relu(flat(x2), hgcn_W[t, 2])
        x3 = spmm_relu(as2(x2m), c1r, c1c, c1v)
        y = spmm(as2(flat(x3)), c2r, c2c, c2v)

        yf = flat(y)
        s1m_in = mm_bias_relu(yf, lg_W[t, 0], lg_b[t, 0].reshape(1, D))
        s1 = spmm(as2(s1m_in), lgr, lgc, lgv)
        s1f = flat(s1)
        s2m_in, ysum = mm_bias_ysum(s1f, lg_W[t, 1], lg_b[t, 1].reshape(1, D),
                                    yf)
        feat_u = spmm_init(as2(s2m_in), lgr, lgc, lgv, as3(ysum))

        dyn_v_parts.append(flat(x3))
        dyn_u_parts.append(flat(feat_u))

    dyn_v = jnp.concatenate(dyn_v_parts, axis=0)
    dyn_u = jnp.concatenate(dyn_u_parts, axis=0)
    return (dyn_u, dyn_v)


# speedup vs baseline: 1.5881x; 1.5881x over previous
import functools

import jax
import jax.numpy as jnp
from jax import lax
from jax.experimental import pallas as pl
from jax.experimental.pallas import tpu as pltpu
from jax.experimental.pallas import tpu_sc as plsc

NC = 2
NS = 16
LN = 16

N = 20000
D = 128
DH = 64
NNZ = 320000
EB = 512
NNZ_PAD = 327680
G_PAD = 20480

_f32 = jnp.float32
_i32 = jnp.int32


def _make_spmm(relu, has_init):
    MS = N // NS
    NCH = (NNZ_PAD // NS) // EB

    mesh = plsc.VectorSubcoreMesh(core_axis_name="c", subcore_axis_name="s")
    scratch = [
        pltpu.VMEM((EB // 128, 128), _i32),
        pltpu.VMEM((EB // 128, 128), _i32),
        pltpu.VMEM((EB,), _f32),
        pltpu.VMEM((EB, DH), _f32),
        pltpu.VMEM((250, DH), _f32),
        pltpu.VMEM_SHARED((N, DH), _f32),
        pltpu.SemaphoreType.DMA,
    ]

    def body(*refs):
        if has_init:
            (x2, rows2, cols2, vals, init, out,
             idxbuf, rowbuf, valbuf, gbuf, obuf, acc, sem) = refs
        else:
            (x2, rows2, cols2, vals, out,
             idxbuf, rowbuf, valbuf, gbuf, obuf, acc, sem) = refs
        c = lax.axis_index("c")
        s = lax.axis_index("s")
        r0 = s * MS

        if has_init:
            pltpu.sync_copy(init.at[pl.ds(r0, MS), c], acc.at[pl.ds(r0, MS)])
        else:
            def zrow(j, _):
                for q in range(4):
                    obuf[j, pl.ds(q * LN, LN)] = jnp.zeros((LN,), _f32)
                return 0
            lax.fori_loop(0, 250, zrow, 0, unroll=8)
            for oi in range(MS // 250):
                pltpu.sync_copy(obuf, acc.at[pl.ds(r0 + oi * 250, 250)])
        plsc.subcore_barrier()

        e128_0 = s * ((NNZ_PAD // NS) // 128)

        def chunk(i, _):
            b128 = e128_0 + i * (EB // 128)
            eb = (s * (NNZ_PAD // NS)) + i * EB
            pltpu.sync_copy(cols2.at[pl.ds(b128, EB // 128)], idxbuf)
            pltpu.sync_copy(rows2.at[pl.ds(b128, EB // 128)], rowbuf)
            pltpu.sync_copy(vals.at[pl.ds(eb, EB)], valbuf)
            def xform(j, _):
                for q in range(8):
                    sl = pl.ds(q * LN, LN)
                    idxbuf[j, sl] = idxbuf[j, sl] * 2 + c
                return 0
            lax.fori_loop(0, EB // 128, xform, 0, unroll=True)
            descs = [
                pltpu.async_copy(x2.at[idxbuf.at[q]],
                                 gbuf.at[pl.ds(q * 128, 128)], sem)
                for q in range(EB // 128)
            ]
            for dd in descs:
                dd.wait()
            def scale(g, _):
                b16 = g * LN
                v16 = valbuf[pl.ds(b16, LN)]
                for jj in range(LN):
                    sp = jnp.broadcast_to(v16[jj], (LN,))
                    e = b16 + jj
                    for q in range(4):
                        sl = pl.ds(q * LN, LN)
                        gbuf[e, sl] = gbuf[e, sl] * sp
                return 0
            lax.fori_loop(0, EB // LN, scale, 0)
            for q in range(EB // 128):
                pltpu.sync_copy(gbuf.at[pl.ds(q * 128, 128)],
                                acc.at[rowbuf.at[q]], add=True)
            return 0

        lax.fori_loop(0, NCH, chunk, 0)
        plsc.subcore_barrier()

        for oi in range(MS // 250):
            ro = r0 + oi * 250
            pltpu.sync_copy(acc.at[pl.ds(ro, 250)], obuf)
            if relu:
                def rl(j, _):
                    for q in range(4):
                        sl = pl.ds(q * LN, LN)
                        obuf[j, sl] = jnp.maximum(obuf[j, sl], 0.0)
                    return 0
                lax.fori_loop(0, 250, rl, 0, unroll=8)
            pltpu.sync_copy(obuf, out.at[pl.ds(ro, 250), c])

    return pl.kernel(
        body,
        out_type=jax.ShapeDtypeStruct((N, 2, DH), _f32),
        mesh=mesh,
        scratch_types=scratch,
        compiler_params=pltpu.CompilerParams(needs_layout_passes=False, use_tc_tiling_on_sc=False),
    )


def _make_gather():
    mesh = plsc.VectorSubcoreMesh(core_axis_name="c", subcore_axis_name="s")
    scratch = [
        pltpu.VMEM((391, 128), _i32),
        pltpu.VMEM((128,), _i32),
        pltpu.VMEM((128,), _i32),
        pltpu.VMEM((128, D), _f32),
        pltpu.SemaphoreType.DMA,
    ]

    def body(E, dcol, gv2, sf, df, dvm, idxbuf, idx2buf, gbuf, sem):
        c = lax.axis_index("c")
        s = lax.axis_index("s")
        w = c * NS + s
        pltpu.sync_copy(dcol, dvm)

        def chunk(i, _):
            row128 = w * 5 + i
            pltpu.sync_copy(gv2.at[row128], idxbuf)
            pltpu.async_copy(E.at[idxbuf], gbuf, sem).wait()
            pltpu.sync_copy(gbuf, sf.at[pl.ds(row128 * 128, 128)])
            for j in range(8):
                sl = pl.ds(j * LN, LN)
                g16 = idxbuf[sl]
                idx2buf[sl] = plsc.load_gather(
                    dvm, [jnp.right_shift(g16, 7),
                          jnp.bitwise_and(g16, 127)])
            pltpu.async_copy(E.at[idx2buf], gbuf, sem).wait()
            pltpu.sync_copy(gbuf, df.at[pl.ds(row128 * 128, 128)])
            return 0

        lax.fori_loop(0, 5, chunk, 0)

    return pl.kernel(
        body,
        out_type=(jax.ShapeDtypeStruct((G_PAD, D), _f32),
                  jax.ShapeDtypeStruct((G_PAD, D), _f32)),
        mesh=mesh,
        scratch_types=scratch,
        compiler_params=pltpu.CompilerParams(needs_layout_passes=False, use_tc_tiling_on_sc=False),
    )


def _pad_dcol(dcol):
    return jnp.concatenate(
        [dcol, jnp.zeros((50048 - 50000,), _i32)]).reshape(391, 128)


_R = 1000


def _fusion_body(sf, df, w1, b1, w2, hg, out):
    a = sf[...]
    b = df[...]
    w1v = w1[...]
    b1v = b1[...]
    w2v = w2[...]
    h0 = jnp.tanh(jnp.dot(a, w1v, preferred_element_type=_f32) + b1v)
    h1 = jnp.tanh(jnp.dot(b, w1v, preferred_element_type=_f32) + b1v)
    s0 = jnp.sum(h0 * w2v.T, axis=1, keepdims=True)
    s1 = jnp.sum(h1 * w2v.T, axis=1, keepdims=True)
    p = jax.nn.sigmoid(s0 - s1)
    fused = p * a + (1.0 - p) * b
    out[...] = jnp.dot(fused, hg[...], preferred_element_type=_f32)


def _make_fusion():
    grid = (N // _R,)
    row = pl.BlockSpec((_R, D), lambda i: (i, 0))
    full = pl.BlockSpec((D, D), lambda i: (0, 0))
    return pl.pallas_call(
        _fusion_body,
        grid=grid,
        in_specs=[row, row, full,
                  pl.BlockSpec((1, D), lambda i: (0, 0)),
                  pl.BlockSpec((D, 1), lambda i: (0, 0)),
                  full],
        out_specs=row,
        out_shape=jax.ShapeDtypeStruct((N, D), _f32),
    )


def _make_mm(relu_x, bias, ysum):
    grid = (N // _R,)
    row = pl.BlockSpec((_R, D), lambda i: (i, 0))
    full = pl.BlockSpec((D, D), lambda i: (0, 0))
    brow = pl.BlockSpec((1, D), lambda i: (0, 0))

    def mm_body(*refs):
        i = 0
        x = refs[i][...]; i += 1
        w = refs[i][...]; i += 1
        bv = None
        if bias:
            bv = refs[i][...]; i += 1
        yv = None
        if ysum:
            yv = jnp.maximum(refs[i][...], 0.0); i += 1
        if relu_x:
            x = jnp.maximum(x, 0.0)
        o = jnp.dot(x, w, preferred_element_type=_f32)
        if bias:
            o = o + bv
        refs[i][...] = o
        if ysum:
            refs[i + 1][...] = yv + x

    in_specs = [row, full]
    if bias:
        in_specs.append(brow)
    if ysum:
        in_specs.append(row)
    out_shape = jax.ShapeDtypeStruct((N, D), _f32)
    if ysum:
        return pl.pallas_call(mm_body, grid=grid, in_specs=in_specs,
                              out_specs=(row, row),
                              out_shape=(out_shape, out_shape))
    return pl.pallas_call(mm_body, grid=grid, in_specs=in_specs,
                          out_specs=row, out_shape=out_shape)


def _pad_edges(rows, cols, vals):
    pad = NNZ_PAD - NNZ
    rows2 = jnp.concatenate([rows, jnp.zeros((pad,), _i32)]).reshape(-1, 128)
    cols2 = jnp.concatenate([cols, jnp.zeros((pad,), _i32)]).reshape(-1, 128)
    valsp = jnp.concatenate([vals, jnp.zeros((pad,), _f32)])
    return rows2, cols2, valsp


def kernel(static_embeddings, fusion_W1, fusion_b1, fusion_W2, fusion_b2,
           hgcn_W, lg_W, lg_b,
           coef1_vals, coef2_vals, lg_vals,
           guid_v, duid_trace_v,
           coef1_rows, coef1_cols, coef2_rows, coef2_cols, lg_rows, lg_cols):
    E = static_embeddings
    T = guid_v.shape[0]

    spmm = _make_spmm(relu=False, has_init=False)
    spmm_relu = _make_spmm(relu=True, has_init=False)
    spmm_init = _make_spmm(relu=False, has_init=True)
    gather = _make_gather()
    fusion = _make_fusion()
    mm_relu = _make_mm(relu_x=True, bias=False, ysum=False)
    mm_bias_relu = _make_mm(relu_x=True, bias=True, ysum=False)
    mm_bias_ysum = _make_mm(relu_x=False, bias=True, ysum=True)

    def as2(x):
        return x.reshape(2 * N, DH)

    def as3(x):
        return x.reshape(N, 2, DH)

    def flat(x):
        return x.reshape(N, D)

    dyn_v_parts = [E]
    dyn_u_parts = [E]
    for t in range(T):
        gvp = jnp.concatenate(
            [guid_v[t], jnp.zeros((G_PAD - N,), _i32)]).reshape(-1, 128)
        dcol = _pad_dcol(duid_trace_v[:, t])
        sfp, dfp = gather(E, dcol, gvp)
        sf, df = sfp[:N], dfp[:N]

        x0 = fusion(sf, df, fusion_W1[t], fusion_b1[t].reshape(1, D),
                    fusion_W2[t], hgcn_W[t, 0])

        c1r, c1c, c1v = _pad_edges(coef1_rows[t], coef1_cols[t], coef1_vals[t])
        c2r, c2c, c2v = _pad_edges(coef2_rows[t], coef2_cols[t], coef2_vals[t])
        lgr, lgc, lgv = _pad_edges(lg_rows[t], lg_cols[t], lg_vals[t])

        x1 = spmm(as2(x0), c1r, c1c, c1v)
        x1m = mm_relu(flat(x1), hgcn_W[t, 1])
        x2 = spmm(as2(x1m), c1r, c1c, c1v)
        x2m = mm_relu(flat(x2), hgcn_W[t, 2])
        x3 = spmm_relu(as2(x2m), c1r, c1c, c1v)
        y = spmm(as2(flat(x3)), c2r, c2c, c2v)

        yf = flat(y)
        s1m_in = mm_bias_relu(yf, lg_W[t, 0], lg_b[t, 0].reshape(1, D))
        s1 = spmm(as2(s1m_in), lgr, lgc, lgv)
        s1f = flat(s1)
        s2m_in, ysum = mm_bias_ysum(s1f, lg_W[t, 1], lg_b[t, 1].reshape(1, D),
                                    yf)
        feat_u = spmm_init(as2(s2m_in), lgr, lgc, lgv, as3(ysum))

        dyn_v_parts.append(flat(x3))
        dyn_u_parts.append(flat(feat_u))

    dyn_v = jnp.concatenate(dyn_v_parts, axis=0)
    dyn_u = jnp.concatenate(dyn_u_parts, axis=0)
    return (dyn_u, dyn_v)


# speedup vs baseline: 2.0899x; 1.3160x over previous
import functools

import jax
import jax.numpy as jnp
from jax import lax
from jax.experimental import pallas as pl
from jax.experimental.pallas import tpu as pltpu
from jax.experimental.pallas import tpu_sc as plsc

NC = 2
NS = 16
LN = 16

N = 20000
D = 128
DH = 64
NNZ = 320000
EB = 256
NNZ_PAD = 327680
G_PAD = 20480

_f32 = jnp.float32
_i32 = jnp.int32


def _make_spmm(relu, has_init):
    MS = N // NS
    ESP = NNZ_PAD // NS
    NCH = ESP // EB
    NQ = EB // 128

    mesh = plsc.VectorSubcoreMesh(core_axis_name="c", subcore_axis_name="s")
    scratch = [
        pltpu.VMEM((4, NQ, 128), _i32),
        pltpu.VMEM((4, NQ, 128), _i32),
        pltpu.VMEM((4, EB), _f32),
        pltpu.VMEM((2, EB, DH), _f32),
        pltpu.VMEM((125, DH), _f32),
        pltpu.VMEM_SHARED((N, DH), _f32),
        pltpu.SemaphoreType.DMA((4,)),
        pltpu.SemaphoreType.DMA((2,)),
        pltpu.SemaphoreType.DMA((2,)),
    ]

    def body(*refs):
        if has_init:
            (x2, rows2, cols2, vals, init, out,
             idxb, rowb, valb, gbuf, obuf, acc, msem, gsem, ssem) = refs
        else:
            (x2, rows2, cols2, vals, out,
             idxb, rowb, valb, gbuf, obuf, acc, msem, gsem, ssem) = refs
        c = lax.axis_index("c")
        s = lax.axis_index("s")
        r0 = s * MS

        if has_init:
            pltpu.sync_copy(init.at[pl.ds(r0, MS), c], acc.at[pl.ds(r0, MS)])
        else:
            def zrow(j, _):
                for q in range(4):
                    obuf[j, pl.ds(q * LN, LN)] = jnp.zeros((LN,), _f32)
                return 0
            lax.fori_loop(0, 125, zrow, 0, unroll=5)
            for oi in range(MS // 125):
                pltpu.sync_copy(obuf, acc.at[pl.ds(r0 + oi * 125, 125)])
        plsc.subcore_barrier()

        e128_0 = s * (ESP // 128)
        eb_0 = s * ESP

        def fire_meta(ci, m):
            pltpu.async_copy(cols2.at[pl.ds(e128_0 + ci * NQ, NQ)],
                             idxb.at[m], msem.at[m])
            pltpu.async_copy(rows2.at[pl.ds(e128_0 + ci * NQ, NQ)],
                             rowb.at[m], msem.at[m])
            pltpu.async_copy(vals.at[pl.ds(eb_0 + ci * EB, EB)],
                             valb.at[m], msem.at[m])

        def wait_meta_fire_gather(ci, m, b):
            pltpu.make_async_copy(cols2.at[pl.ds(e128_0 + ci * NQ, NQ)],
                                  idxb.at[m], msem.at[m]).wait()
            pltpu.make_async_copy(rows2.at[pl.ds(e128_0 + ci * NQ, NQ)],
                                  rowb.at[m], msem.at[m]).wait()
            pltpu.make_async_copy(vals.at[pl.ds(eb_0 + ci * EB, EB)],
                                  valb.at[m], msem.at[m]).wait()
            def xform(j, _):
                for q in range(NQ):
                    sl = pl.ds(j * LN, LN)
                    idxb[m, q, sl] = idxb[m, q, sl] * 2 + c
                return 0
            lax.fori_loop(0, 128 // LN, xform, 0, unroll=True)
            for q in range(NQ):
                pltpu.async_copy(x2.at[idxb.at[m, q]],
                                 gbuf.at[b, pl.ds(q * 128, 128)], gsem.at[b])

        def drain_gather(m, b):
            for q in range(NQ):
                pltpu.make_async_copy(x2.at[idxb.at[m, q]],
                                      gbuf.at[b, pl.ds(q * 128, 128)],
                                      gsem.at[b]).wait()

        def scale(m, b):
            def sc(g, _):
                b16 = g * LN
                v16 = valb[m, pl.ds(b16, LN)]
                for jj in range(LN):
                    sp = jnp.broadcast_to(v16[jj], (LN,))
                    e = b16 + jj
                    for q in range(4):
                        sl = pl.ds(q * LN, LN)
                        gbuf[b, e, sl] = gbuf[b, e, sl] * sp
                return 0
            lax.fori_loop(0, EB // LN, sc, 0)

        def fire_scatter(m, b):
            for q in range(NQ):
                pltpu.async_copy(gbuf.at[b, pl.ds(q * 128, 128)],
                                 acc.at[rowb.at[m, q]], ssem.at[b], add=True)

        def drain_scatter(m, b):
            for q in range(NQ):
                pltpu.make_async_copy(gbuf.at[b, pl.ds(q * 128, 128)],
                                      acc.at[rowb.at[m, q]],
                                      ssem.at[b]).wait()

        def step(ci, k, drain_prev, fire_m, fire_g):
            b = k % 2
            m = k % 4
            drain_gather(m, b)
            scale(m, b)
            if drain_prev:
                drain_scatter((k - 1) % 4, 1 - b)
            if fire_m:
                fire_meta(ci + 2, (k + 2) % 4)
            if fire_g:
                wait_meta_fire_gather(ci + 1, (k + 1) % 4, 1 - b)
            fire_scatter(m, b)

        fire_meta(0, 0)
        fire_meta(1, 1)
        wait_meta_fire_gather(0, 0, 0)
        step(0, 0, False, True, True)
        for k in range(1, 4):
            step(k, k, True, True, True)

        def group(g, _):
            base = g * 4
            for k in range(4):
                step(base + k, k, True, True, True)
            return 0
        lax.fori_loop(1, NCH // 4 - 1, group, 0)

        base = NCH - 4
        step(base + 0, 0, True, True, True)
        step(base + 1, 1, True, True, True)
        step(base + 2, 2, True, False, True)
        step(base + 3, 3, True, False, False)
        drain_scatter(3, 1)
        plsc.subcore_barrier()

        for oi in range(MS // 125):
            ro = r0 + oi * 125
            pltpu.sync_copy(acc.at[pl.ds(ro, 125)], obuf)
            if relu:
                def rl(j, _):
                    for q in range(4):
                        sl = pl.ds(q * LN, LN)
                        obuf[j, sl] = jnp.maximum(obuf[j, sl], 0.0)
                    return 0
                lax.fori_loop(0, 125, rl, 0, unroll=5)
            pltpu.sync_copy(obuf, out.at[pl.ds(ro, 125), c])

    return pl.kernel(
        body,
        out_type=jax.ShapeDtypeStruct((N, 2, DH), _f32),
        mesh=mesh,
        scratch_types=scratch,
        compiler_params=pltpu.CompilerParams(needs_layout_passes=False, use_tc_tiling_on_sc=False),
    )


def _make_gather():
    mesh = plsc.VectorSubcoreMesh(core_axis_name="c", subcore_axis_name="s")
    scratch = [
        pltpu.VMEM((391, 128), _i32),
        pltpu.VMEM((128,), _i32),
        pltpu.VMEM((128,), _i32),
        pltpu.VMEM((128, D), _f32),
        pltpu.SemaphoreType.DMA,
    ]

    def body(E, dcol, gv2, sf, df, dvm, idxbuf, idx2buf, gbuf, sem):
        c = lax.axis_index("c")
        s = lax.axis_index("s")
        w = c * NS + s
        pltpu.sync_copy(dcol, dvm)

        def chunk(i, _):
            row128 = w * 5 + i
            pltpu.sync_copy(gv2.at[row128], idxbuf)
            pltpu.async_copy(E.at[idxbuf], gbuf, sem).wait()
            pltpu.sync_copy(gbuf, sf.at[pl.ds(row128 * 128, 128)])
            for j in range(8):
                sl = pl.ds(j * LN, LN)
                g16 = idxbuf[sl]
                idx2buf[sl] = plsc.load_gather(
                    dvm, [jnp.right_shift(g16, 7),
                          jnp.bitwise_and(g16, 127)])
            pltpu.async_copy(E.at[idx2buf], gbuf, sem).wait()
            pltpu.sync_copy(gbuf, df.at[pl.ds(row128 * 128, 128)])
            return 0

        lax.fori_loop(0, 5, chunk, 0)

    return pl.kernel(
        body,
        out_type=(jax.ShapeDtypeStruct((G_PAD, D), _f32),
                  jax.ShapeDtypeStruct((G_PAD, D), _f32)),
        mesh=mesh,
        scratch_types=scratch,
        compiler_params=pltpu.CompilerParams(needs_layout_passes=False, use_tc_tiling_on_sc=False),
    )


def _pad_dcol(dcol):
    return jnp.concatenate(
        [dcol, jnp.zeros((50048 - 50000,), _i32)]).reshape(391, 128)


_R = 1000


def _fusion_body(sf, df, w1, b1, w2, hg, out):
    a = sf[...]
    b = df[...]
    w1v = w1[...]
    b1v = b1[...]
    w2v = w2[...]
    h0 = jnp.tanh(jnp.dot(a, w1v, preferred_element_type=_f32) + b1v)
    h1 = jnp.tanh(jnp.dot(b, w1v, preferred_element_type=_f32) + b1v)
    s0 = jnp.sum(h0 * w2v.T, axis=1, keepdims=True)
    s1 = jnp.sum(h1 * w2v.T, axis=1, keepdims=True)
    p = jax.nn.sigmoid(s0 - s1)
    fused = p * a + (1.0 - p) * b
    out[...] = jnp.dot(fused, hg[...], preferred_element_type=_f32)


def _make_fusion():
    grid = (N // _R,)
    row = pl.BlockSpec((_R, D), lambda i: (i, 0))
    full = pl.BlockSpec((D, D), lambda i: (0, 0))
    return pl.pallas_call(
        _fusion_body,
        grid=grid,
        in_specs=[row, row, full,
                  pl.BlockSpec((1, D), lambda i: (0, 0)),
                  pl.BlockSpec((D, 1), lambda i: (0, 0)),
                  full],
        out_specs=row,
        out_shape=jax.ShapeDtypeStruct((N, D), _f32),
    )


def _make_mm(relu_x, bias, ysum):
    grid = (N // _R,)
    row = pl.BlockSpec((_R, D), lambda i: (i, 0))
    full = pl.BlockSpec((D, D), lambda i: (0, 0))
    brow = pl.BlockSpec((1, D), lambda i: (0, 0))

    def mm_body(*refs):
        i = 0
        x = refs[i][...]; i += 1
        w = refs[i][...]; i += 1
        bv = None
        if bias:
            bv = refs[i][...]; i += 1
        yv = None
        if ysum:
            yv = jnp.maximum(refs[i][...], 0.0); i += 1
        if relu_x:
            x = jnp.maximum(x, 0.0)
        o = jnp.dot(x, w, preferred_element_type=_f32)
        if bias:
            o = o + bv
        refs[i][...] = o
        if ysum:
            refs[i + 1][...] = yv + x

    in_specs = [row, full]
    if bias:
        in_specs.append(brow)
    if ysum:
        in_specs.append(row)
    out_shape = jax.ShapeDtypeStruct((N, D), _f32)
    if ysum:
        return pl.pallas_call(mm_body, grid=grid, in_specs=in_specs,
                              out_specs=(row, row),
                              out_shape=(out_shape, out_shape))
    return pl.pallas_call(mm_body, grid=grid, in_specs=in_specs,
                          out_specs=row, out_shape=out_shape)


def _pad_edges(rows, cols, vals):
    pad = NNZ_PAD - NNZ
    rows2 = jnp.concatenate([rows, jnp.zeros((pad,), _i32)]).reshape(-1, 128)
    cols2 = jnp.concatenate([cols, jnp.zeros((pad,), _i32)]).reshape(-1, 128)
    valsp = jnp.concatenate([vals, jnp.zeros((pad,), _f32)])
    return rows2, cols2, valsp


def kernel(static_embeddings, fusion_W1, fusion_b1, fusion_W2, fusion_b2,
           hgcn_W, lg_W, lg_b,
           coef1_vals, coef2_vals, lg_vals,
           guid_v, duid_trace_v,
           coef1_rows, coef1_cols, coef2_rows, coef2_cols, lg_rows, lg_cols):
    E = static_embeddings
    T = guid_v.shape[0]

    spmm = _make_spmm(relu=False, has_init=False)
    spmm_relu = _make_spmm(relu=True, has_init=False)
    spmm_init = _make_spmm(relu=False, has_init=True)
    gather = _make_gather()
    fusion = _make_fusion()
    mm_relu = _make_mm(relu_x=True, bias=False, ysum=False)
    mm_bias_relu = _make_mm(relu_x=True, bias=True, ysum=False)
    mm_bias_ysum = _make_mm(relu_x=False, bias=True, ysum=True)

    def as2(x):
        return x.reshape(2 * N, DH)

    def as3(x):
        return x.reshape(N, 2, DH)

    def flat(x):
        return x.reshape(N, D)

    dyn_v_parts = [E]
    dyn_u_parts = [E]
    for t in range(T):
        gvp = jnp.concatenate(
            [guid_v[t], jnp.zeros((G_PAD - N,), _i32)]).reshape(-1, 128)
        dcol = _pad_dcol(duid_trace_v[:, t])
        sfp, dfp = gather(E, dcol, gvp)
        sf, df = sfp[:N], dfp[:N]

        x0 = fusion(sf, df, fusion_W1[t], fusion_b1[t].reshape(1, D),
                    fusion_W2[t], hgcn_W[t, 0])

        c1r, c1c, c1v = _pad_edges(coef1_rows[t], coef1_cols[t], coef1_vals[t])
        c2r, c2c, c2v = _pad_edges(coef2_rows[t], coef2_cols[t], coef2_vals[t])
        lgr, lgc, lgv = _pad_edges(lg_rows[t], lg_cols[t], lg_vals[t])

        x1 = spmm(as2(x0), c1r, c1c, c1v)
        x1m = mm_relu(flat(x1), hgcn_W[t, 1])
        x2 = spmm(as2(x1m), c1r, c1c, c1v)
        x2m = mm_relu(flat(x2), hgcn_W[t, 2])
        x3 = spmm_relu(as2(x2m), c1r, c1c, c1v)
        y = spmm(as2(flat(x3)), c2r, c2c, c2v)

        yf = flat(y)
        s1m_in = mm_bias_relu(yf, lg_W[t, 0], lg_b[t, 0].reshape(1, D))
        s1 = spmm(as2(s1m_in), lgr, lgc, lgv)
        s1f = flat(s1)
        s2m_in, ysum = mm_bias_ysum(s1f, lg_W[t, 1], lg_b[t, 1].reshape(1, D),
                                    yf)
        feat_u = spmm_init(as2(s2m_in), lgr, lgc, lgv, as3(ysum))

        dyn_v_parts.append(flat(x3))
        dyn_u_parts.append(flat(feat_u))

    dyn_v = jnp.concatenate(dyn_v_parts, axis=0)
    dyn_u = jnp.concatenate(dyn_u_parts, axis=0)
    return (dyn_u, dyn_v)


# speedup vs baseline: 2.5960x; 1.2422x over previous
import jax
import jax.numpy as jnp
from jax import lax
from jax.experimental import pallas as pl
from jax.experimental.pallas import tpu as pltpu
from jax.experimental.pallas import tpu_sc as plsc

NC = 2
NS = 16
LN = 16

N = 20000
D = 128
DH = 64
NNZ = 320000
EB = 256
NNZ_PAD = 327680
G_PAD = 20480

_f32 = jnp.float32
_bf16 = jnp.bfloat16
_i32 = jnp.int32

_SC_PARAMS = pltpu.CompilerParams(needs_layout_passes=False,
                                  use_tc_tiling_on_sc=False)


def _make_spmm(relu, has_init, emit_bf16):
    MS = N // NS
    ESP = NNZ_PAD // NS
    NCH = ESP // EB
    NQ = 2

    mesh = plsc.VectorSubcoreMesh(core_axis_name="c", subcore_axis_name="s")
    scratch = [
        pltpu.VMEM((4, EB), _i32),
        pltpu.VMEM((4, NQ, 128), _i32),
        pltpu.VMEM((4, EB), _f32),
        pltpu.VMEM((2, EB, DH), _bf16),
        pltpu.VMEM((256, DH), _f32),
    ]
    if emit_bf16:
        scratch.append(pltpu.VMEM((250, DH), _bf16))
    scratch += [
        pltpu.VMEM_SHARED((N, DH), _f32),
        pltpu.SemaphoreType.DMA((4,)),
        pltpu.SemaphoreType.DMA((2,)),
        pltpu.SemaphoreType.DMA,
    ]

    def body(*refs):
        it = iter(refs)
        x2 = next(it); rows2 = next(it); cols1 = next(it); vals = next(it)
        init = next(it) if has_init else None
        out = next(it)
        outbf = next(it) if emit_bf16 else None
        idxb = next(it); rowb = next(it); valb = next(it)
        gbuf = next(it); sbuf = next(it)
        bbuf = next(it) if emit_bf16 else None
        acc = next(it)
        msem = next(it); gsem = next(it); ssem = next(it)

        c = lax.axis_index("c")
        s = lax.axis_index("s")
        r0 = s * MS

        if has_init:
            pltpu.sync_copy(init.at[pl.ds(r0, MS), c], acc.at[pl.ds(r0, MS)])
        else:
            def zrow(j, _):
                for q in range(4):
                    sbuf[j, pl.ds(q * LN, LN)] = jnp.zeros((LN,), _f32)
                return 0
            lax.fori_loop(0, 250, zrow, 0, unroll=8)
            for oi in range(MS // 250):
                pltpu.sync_copy(sbuf.at[pl.ds(0, 250)],
                                acc.at[pl.ds(r0 + oi * 250, 250)])
        plsc.subcore_barrier()

        e128_0 = s * (ESP // 128)
        eb_0 = s * ESP

        def fire_meta(ci, m):
            pltpu.async_copy(cols1.at[pl.ds(eb_0 + ci * EB, EB)],
                             idxb.at[m], msem.at[m])
            pltpu.async_copy(rows2.at[pl.ds(e128_0 + ci * NQ, NQ)],
                             rowb.at[m], msem.at[m])
            pltpu.async_copy(vals.at[pl.ds(eb_0 + ci * EB, EB)],
                             valb.at[m], msem.at[m])

        def wait_meta_fire_gather(ci, m, b):
            pltpu.make_async_copy(cols1.at[pl.ds(eb_0 + ci * EB, EB)],
                                  idxb.at[m], msem.at[m]).wait()
            pltpu.make_async_copy(rows2.at[pl.ds(e128_0 + ci * NQ, NQ)],
                                  rowb.at[m], msem.at[m]).wait()
            pltpu.make_async_copy(vals.at[pl.ds(eb_0 + ci * EB, EB)],
                                  valb.at[m], msem.at[m]).wait()
            def xform(j, _):
                sl = pl.ds(j * LN, LN)
                idxb[m, sl] = idxb[m, sl] * 2 + c
                return 0
            lax.fori_loop(0, EB // LN, xform, 0, unroll=True)
            pltpu.async_copy(x2.at[idxb.at[m]], gbuf.at[b], gsem.at[b])

        def drain_gather(m, b):
            pltpu.make_async_copy(x2.at[idxb.at[m]], gbuf.at[b],
                                  gsem.at[b]).wait()

        def scale(m, b):
            def sc(g, _):
                b16 = g * LN
                v16 = valb[m, pl.ds(b16, LN)]
                for jj in range(LN):
                    sp = jnp.broadcast_to(v16[jj], (LN,))
                    e = b16 + jj
                    for q in range(2):
                        r = gbuf[b, e, pl.ds(q * 32, 32)]
                        lo, hi = plsc.unpack(
                            r, format=plsc.PackFormat.INTERLEAVED)
                        sbuf[e, pl.ds(q * 32, LN)] = lo * sp
                        sbuf[e, pl.ds(q * 32 + LN, LN)] = hi * sp
                return 0
            lax.fori_loop(0, EB // LN, sc, 0)

        def fire_scatter(m):
            for q in range(NQ):
                pltpu.async_copy(sbuf.at[pl.ds(q * 128, 128)],
                                 acc.at[rowb.at[m, q]], ssem, add=True)

        def drain_scatter(m):
            for q in range(NQ):
                pltpu.make_async_copy(sbuf.at[pl.ds(q * 128, 128)],
                                      acc.at[rowb.at[m, q]], ssem).wait()

        def step(ci, k, drain_prev, fire_m, fire_g):
            b = k % 2
            m = k % 4
            drain_gather(m, b)
            if drain_prev:
                drain_scatter((k - 1) % 4)
            scale(m, b)
            if fire_m:
                fire_meta(ci + 2, (k + 2) % 4)
            if fire_g:
                wait_meta_fire_gather(ci + 1, (k + 1) % 4, 1 - b)
            fire_scatter(m)

        fire_meta(0, 0)
        fire_meta(1, 1)
        wait_meta_fire_gather(0, 0, 0)
        step(0, 0, False, True, True)
        for k in range(1, 4):
            step(k, k, True, True, True)

        def group(g, _):
            base = g * 4
            for k in range(4):
                step(base + k, k, True, True, True)
            return 0
        lax.fori_loop(1, NCH // 4 - 1, group, 0)

        base = NCH - 4
        step(base + 0, 0, True, True, True)
        step(base + 1, 1, True, True, True)
        step(base + 2, 2, True, False, True)
        step(base + 3, 3, True, False, False)
        drain_scatter(3)
        plsc.subcore_barrier()

        for oi in range(MS // 250):
            ro = r0 + oi * 250
            pltpu.sync_copy(acc.at[pl.ds(ro, 250)], sbuf.at[pl.ds(0, 250)])
            if relu:
                def rl(j, _):
                    for q in range(4):
                        sl = pl.ds(q * LN, LN)
                        sbuf[j, sl] = jnp.maximum(sbuf[j, sl], 0.0)
                    return 0
                lax.fori_loop(0, 250, rl, 0, unroll=8)
            if emit_bf16:
                def pk(j, _):
                    for q in range(2):
                        a = sbuf[j, pl.ds(q * 32, LN)]
                        bq = sbuf[j, pl.ds(q * 32 + LN, LN)]
                        bbuf[j, pl.ds(q * 32, 32)] = plsc.pack(
                            a, bq, format=plsc.PackFormat.INTERLEAVED)
                    return 0
                lax.fori_loop(0, 250, pk, 0, unroll=8)
                pltpu.sync_copy(bbuf, outbf.at[pl.ds(ro, 250), c])
            pltpu.sync_copy(sbuf.at[pl.ds(0, 250)],
                            out.at[pl.ds(ro, 250), c])

    out_type = [jax.ShapeDtypeStruct((N, 2, DH), _f32)]
    if emit_bf16:
        out_type.append(jax.ShapeDtypeStruct((N, 2, DH), _bf16))

    return pl.kernel(
        body,
        out_type=tuple(out_type) if len(out_type) > 1 else out_type[0],
        mesh=mesh,
        scratch_types=scratch,
        compiler_params=_SC_PARAMS,
    )


def _make_gather():
    mesh = plsc.VectorSubcoreMesh(core_axis_name="c", subcore_axis_name="s")
    scratch = [
        pltpu.VMEM((391, 128), _i32),
        pltpu.VMEM((128,), _i32),
        pltpu.VMEM((128,), _i32),
        pltpu.VMEM((128, D), _f32),
        pltpu.SemaphoreType.DMA,
    ]

    def body(E, dcol, gv2, sf, df, dvm, idxbuf, idx2buf, gbuf, sem):
        c = lax.axis_index("c")
        s = lax.axis_index("s")
        w = c * NS + s
        pltpu.sync_copy(dcol, dvm)

        def chunk(i, _):
            row128 = w * 5 + i
            pltpu.sync_copy(gv2.at[row128], idxbuf)
            pltpu.async_copy(E.at[idxbuf], gbuf, sem).wait()
            pltpu.sync_copy(gbuf, sf.at[pl.ds(row128 * 128, 128)])
            for j in range(8):
                sl = pl.ds(j * LN, LN)
                g16 = idxbuf[sl]
                idx2buf[sl] = plsc.load_gather(
                    dvm, [jnp.right_shift(g16, 7),
                          jnp.bitwise_and(g16, 127)])
            pltpu.async_copy(E.at[idx2buf], gbuf, sem).wait()
            pltpu.sync_copy(gbuf, df.at[pl.ds(row128 * 128, 128)])
            return 0

        lax.fori_loop(0, 5, chunk, 0)

    return pl.kernel(
        body,
        out_type=(jax.ShapeDtypeStruct((G_PAD, D), _f32),
                  jax.ShapeDtypeStruct((G_PAD, D), _f32)),
        mesh=mesh,
        scratch_types=scratch,
        compiler_params=_SC_PARAMS,
    )


_R = 2000


def _swizzle_bf16(x):
    return x.astype(_bf16)


def _fusion_body(sf, df, w1, b1, w2, hg, outbf):
    a = sf[...]
    b = df[...]
    w1v = w1[...]
    b1v = b1[...]
    w2v = w2[...]
    h0 = jnp.tanh(jnp.dot(a, w1v, preferred_element_type=_f32) + b1v)
    h1 = jnp.tanh(jnp.dot(b, w1v, preferred_element_type=_f32) + b1v)
    s0 = jnp.sum(h0 * w2v.T, axis=1, keepdims=True)
    s1 = jnp.sum(h1 * w2v.T, axis=1, keepdims=True)
    p = jax.nn.sigmoid(s0 - s1)
    fused = p * a + (1.0 - p) * b
    outbf[...] = _swizzle_bf16(jnp.dot(fused, hg[...],
                                       preferred_element_type=_f32))


def _make_fusion():
    grid = (N // _R,)
    row = pl.BlockSpec((_R, D), lambda i: (i, 0))
    full = pl.BlockSpec((D, D), lambda i: (0, 0))
    return pl.pallas_call(
        _fusion_body,
        grid=grid,
        in_specs=[row, row, full,
                  pl.BlockSpec((1, D), lambda i: (0, 0)),
                  pl.BlockSpec((D, 1), lambda i: (0, 0)),
                  full],
        out_specs=row,
        out_shape=jax.ShapeDtypeStruct((N, D), _bf16),
    )


def _make_mm(relu_x, bias, ysum, out_f32=False):
    grid = (N // _R,)
    row = pl.BlockSpec((_R, D), lambda i: (i, 0))
    full = pl.BlockSpec((D, D), lambda i: (0, 0))
    brow = pl.BlockSpec((1, D), lambda i: (0, 0))

    def mm_body(*refs):
        i = 0
        x = refs[i][...]; i += 1
        w = refs[i][...]; i += 1
        bv = None
        if bias:
            bv = refs[i][...]; i += 1
        yv = None
        if ysum:
            yv = jnp.maximum(refs[i][...], 0.0); i += 1
        if relu_x:
            x = jnp.maximum(x, 0.0)
        o = jnp.dot(x, w, preferred_element_type=_f32)
        if bias:
            o = o + bv
        refs[i][...] = o if out_f32 else _swizzle_bf16(o)
        if ysum:
            refs[i + 1][...] = yv + x

    in_specs = [row, full]
    if bias:
        in_specs.append(brow)
    if ysum:
        in_specs.append(row)
    o1 = jax.ShapeDtypeStruct((N, D), _f32 if out_f32 else _bf16)
    if ysum:
        return pl.pallas_call(mm_body, grid=grid, in_specs=in_specs,
                              out_specs=(row, row),
                              out_shape=(o1,
                                         jax.ShapeDtypeStruct((N, D), _f32)))
    return pl.pallas_call(mm_body, grid=grid, in_specs=in_specs,
                          out_specs=row, out_shape=o1)


def _pad_edges(rows, cols, vals):
    pad = NNZ_PAD - NNZ
    rows2 = jnp.concatenate([rows, jnp.zeros((pad,), _i32)]).reshape(-1, 128)
    cols1 = jnp.concatenate([cols, jnp.zeros((pad,), _i32)])
    valsp = jnp.concatenate([vals, jnp.zeros((pad,), _f32)])
    return rows2, cols1, valsp


def _perm_matrix():
    import numpy as _np
    p = _np.zeros((D, D), _np.float32)
    for q in range(4):
        for i in range(LN):
            p[32 * q + i, 32 * q + 2 * i] = 1.0
            p[32 * q + LN + i, 32 * q + 2 * i + 1] = 1.0
    return jnp.asarray(p)


def _pad_dcol(dcol):
    return jnp.concatenate(
        [dcol, jnp.zeros((50048 - 50000,), _i32)]).reshape(391, 128)


def kernel(static_embeddings, fusion_W1, fusion_b1, fusion_W2, fusion_b2,
           hgcn_W, lg_W, lg_b,
           coef1_vals, coef2_vals, lg_vals,
           guid_v, duid_trace_v,
           coef1_rows, coef1_cols, coef2_rows, coef2_cols, lg_rows, lg_cols):
    E = static_embeddings
    T = guid_v.shape[0]

    spmm = _make_spmm(relu=False, has_init=False, emit_bf16=False)
    spmm_relu_bf = _make_spmm(relu=True, has_init=False, emit_bf16=True)
    spmm_init = _make_spmm(relu=False, has_init=True, emit_bf16=False)
    gather = _make_gather()
    fusion = _make_fusion()
    mm_relu = _make_mm(relu_x=True, bias=False, ysum=False)
    mm_bias_relu = _make_mm(relu_x=True, bias=True, ysum=False)
    mm_bias_ysum = _make_mm(relu_x=False, bias=True, ysum=True)
    mm_unperm = _make_mm(relu_x=False, bias=False, ysum=False, out_f32=True)

    P = _perm_matrix()
    Pt = P.T

    def as2(x):
        return x.reshape(2 * N, DH)

    def as3(x):
        return x.reshape(N, 2, DH)

    def flat(x):
        return x.reshape(N, D)

    dyn_v_parts = [E]
    dyn_u_parts = [E]
    for t in range(T):
        gvp = jnp.concatenate(
            [guid_v[t], jnp.zeros((G_PAD - N,), _i32)]).reshape(-1, 128)
        dcol = _pad_dcol(duid_trace_v[:, t])
        sfp, dfp = gather(E, dcol, gvp)
        sf, df = sfp[:N], dfp[:N]

        x0bf = fusion(sf, df, fusion_W1[t], fusion_b1[t].reshape(1, D),
                      fusion_W2[t], hgcn_W[t, 0] @ P)

        c1r, c1c, c1v = _pad_edges(coef1_rows[t], coef1_cols[t], coef1_vals[t])
        c2r, c2c, c2v = _pad_edges(coef2_rows[t], coef2_cols[t], coef2_vals[t])
        lgr, lgc, lgv = _pad_edges(lg_rows[t], lg_cols[t], lg_vals[t])

        x1 = spmm(as2(x0bf), c1r, c1c, c1v)
        x1mbf = mm_relu(flat(x1), Pt @ hgcn_W[t, 1] @ P)
        x2 = spmm(as2(x1mbf), c1r, c1c, c1v)
        x2mbf = mm_relu(flat(x2), Pt @ hgcn_W[t, 2] @ P)
        x3p, x3bf = spmm_relu_bf(as2(x2mbf), c1r, c1c, c1v)
        y = spmm(as2(x3bf.reshape(N, D)), c2r, c2c, c2v)

        yf = flat(y)
        s1mbf = mm_bias_relu(yf, Pt @ lg_W[t, 0] @ P,
                             (lg_b[t, 0] @ P).reshape(1, D))
        s1 = spmm(as2(s1mbf), lgr, lgc, lgv)
        s2mbf, ysum = mm_bias_ysum(flat(s1), Pt @ lg_W[t, 1] @ P,
                                   (lg_b[t, 1] @ P).reshape(1, D), yf)
        feat_up = spmm_init(as2(s2mbf), lgr, lgc, lgv, as3(ysum))

        dyn_v_parts.append(mm_unperm(flat(x3p), Pt))
        dyn_u_parts.append(mm_unperm(flat(feat_up), Pt))

    dyn_v = jnp.concatenate(dyn_v_parts, axis=0)
    dyn_u = jnp.concatenate(dyn_u_parts, axis=0)
    return (dyn_u, dyn_v)
